# Initial kernel scaffold; baseline (speedup 1.0000x reference)
import jax
import jax.numpy as jnp
from jax import lax
from jax.experimental import pallas as pl
from jax.experimental.pallas import tpu as pltpu
from jax.experimental.pallas import tpu_sc as plsc

T = 512
V = 32768
B = 8
D = 64
W_ASR = 0.7
W_LM = 0.3
NEG = -1e30

NS = 16
VS = V // NS
NJ = VS // 16


def _asr_body(x_ref, o_ref):
    x = x_ref[...]
    m = jnp.max(x, axis=1, keepdims=True)
    sh = x - m
    lse = jnp.log(jnp.sum(jnp.exp(sh), axis=1, keepdims=True))
    o_ref[...] = W_ASR * (sh - lse)


def _asr_prep(asr):
    RB = 64
    return pl.pallas_call(
        _asr_body,
        grid=(T // RB,),
        in_specs=[pl.BlockSpec((RB, V), lambda i: (i, 0))],
        out_specs=pl.BlockSpec((RB, V), lambda i: (i, 0)),
        out_shape=jax.ShapeDtypeStruct((T, V), jnp.float32),
    )(asr)


def _tab_body(emb_ref, proj_ref, o_ref):
    logits = jnp.dot(emb_ref[...], proj_ref[...],
                     preferred_element_type=jnp.float32)
    m = jnp.max(logits, axis=1, keepdims=True)
    sh = logits - m
    lse = jnp.log(jnp.sum(jnp.exp(sh), axis=1, keepdims=True))
    o_ref[...] = W_LM * (sh - lse)


def _lm_table(emb, proj):
    RB = 64
    return pl.pallas_call(
        _tab_body,
        grid=(V // RB,),
        in_specs=[pl.BlockSpec((RB, D), lambda i: (i, 0)),
                  pl.BlockSpec((D, V), lambda i: (0, 0))],
        out_specs=pl.BlockSpec((RB, V), lambda i: (i, 0)),
        out_shape=jax.ShapeDtypeStruct((V, V), jnp.float32),
    )(emb, proj)


def _sc_body(tab_ref, asr_ref,
             fs_ref, tok_ref, par_ref,
             idx_v, rows_v, asr_v, wbuf, cval, cidx,
             tokbuf, parbuf, fsbuf, pubv, pubi,
             shv, shi,
             sem):
    wid = lax.axis_index("s")
    lane = lax.iota(jnp.int32, 16)
    lane_lt8 = lane < 8
    negv = jnp.full((16,), NEG, jnp.float32)
    zeroi = jnp.zeros((16,), jnp.int32)
    col0 = wid * VS

    score0 = jnp.where(lane == 0, jnp.float32(0.0), jnp.float32(NEG))
    tok0 = zeroi

    def step(t, carry):
        score, tok = carry

        idx_v[...] = tok * NS + wid
        lm_cp = pltpu.make_async_copy(tab_ref.at[idx_v], rows_v, sem)
        lm_cp.start()
        asr_cp = pltpu.make_async_copy(
            asr_ref.at[pl.ds((t * NS + wid) * VS, VS)], asr_v, sem)
        asr_cp.start()
        lm_cp.wait()
        asr_cp.wait()

        sbs = [jnp.take(score, jnp.full((16,), b, jnp.int32),
                        mode="promise_in_bounds") for b in range(B)]

        def p1(j, m):
            a = asr_v[pl.ds(j * 16, 16)]
            for b in range(B):
                w = (a + rows_v[b, pl.ds(j * 16, 16)]) + sbs[b]
                wbuf[pl.ds(b * VS + j * 16, 16)] = w
                m = jnp.maximum(m, w)
            return m

        m = lax.fori_loop(0, NJ, p1, negv)

        msort, _ = plsc.sort_key_val(m, lane, descending=True)
        thr = jnp.take(msort, jnp.full((16,), 7, jnp.int32),
                       mode="promise_in_bounds")

        def p2(j, off):
            for b in range(B):
                w = wbuf[pl.ds(b * VS + j * 16, 16)]
                sel = w >= thr
                iv = lane + (b * V + col0 + j * 16)
                plsc.store_compressed(cval.at[pl.ds(off, 16)], w, mask=sel)
                plsc.store_compressed(cidx.at[pl.ds(off, 16)], iv, mask=sel)
                off = off + jnp.max(plsc.all_reduce_population_count(sel))
            return off

        off = lax.fori_loop(0, NJ, p2, jnp.int32(0))
        cval[pl.ds(off, 16)] = negv

        def mrg(k, cur):
            cv, ci = cur
            vs, ivs = plsc.sort_key_val(cval[pl.ds(k * 16, 16)],
                                        cidx[pl.ds(k * 16, 16)],
                                        descending=True)
            rv = lax.rev(vs, (0,))
            ri = lax.rev(ivs, (0,))
            keep = cv >= rv
            return plsc.sort_key_val(jnp.where(keep, cv, rv),
                                     jnp.where(keep, ci, ri),
                                     descending=True)

        nv = (off + 15) >> 4
        cv, ci = lax.fori_loop(0, nv, mrg, (negv, zeroi))

        pubv[...] = cv
        pubi[...] = ci
        pltpu.sync_copy(pubv, shv.at[wid])
        pltpu.sync_copy(pubi, shi.at[wid])
        plsc.subcore_barrier()
        pltpu.sync_copy(shv, cval.at[pl.ds(0, NS * 16)])
        pltpu.sync_copy(shi, cidx.at[pl.ds(0, NS * 16)])
        plsc.subcore_barrier()

        def gmrg(k, cur):
            cv2, ci2 = cur
            vs = cval[pl.ds(k * 16, 16)]
            ivs = cidx[pl.ds(k * 16, 16)]
            rv = lax.rev(vs, (0,))
            ri = lax.rev(ivs, (0,))
            keep = cv2 >= rv
            return plsc.sort_key_val(jnp.where(keep, cv2, rv),
                                     jnp.where(keep, ci2, ri),
                                     descending=True)

        gv, gi = lax.fori_loop(0, NS, gmrg, (negv, zeroi))

        par = lax.shift_right_arithmetic(gi, 15)
        tokn = jnp.bitwise_and(gi, V - 1)
        plsc.store_compressed(tokbuf.at[pl.ds(t * B, 16)], tokn, mask=lane_lt8)
        plsc.store_compressed(parbuf.at[pl.ds(t * B, 16)], par, mask=lane_lt8)
        return (gv, tokn)

    score, _ = lax.fori_loop(0, T, step, (score0, tok0))

    @pl.when(wid == 0)
    def _():
        fsbuf[...] = score
        pltpu.sync_copy(fsbuf, fs_ref)
        pltpu.sync_copy(tokbuf.at[pl.ds(0, T * B)], tok_ref)
        pltpu.sync_copy(parbuf.at[pl.ds(0, T * B)], par_ref)


def _beam_sc(tab, asr_w):
    mesh = plsc.VectorSubcoreMesh(core_axis_name="c", subcore_axis_name="s",
                                  num_cores=1)
    fn = pl.kernel(
        _sc_body,
        out_type=[
            jax.ShapeDtypeStruct((16,), jnp.float32),
            jax.ShapeDtypeStruct((T * B,), jnp.int32),
            jax.ShapeDtypeStruct((T * B,), jnp.int32),
        ],
        mesh=mesh,
        scratch_types=[
            pltpu.VMEM((16,), jnp.int32),
            pltpu.VMEM((16, VS), jnp.float32),
            pltpu.VMEM((VS,), jnp.float32),
            pltpu.VMEM((B * VS,), jnp.float32),
            pltpu.VMEM((B * VS + 16,), jnp.float32),
            pltpu.VMEM((B * VS + 16,), jnp.int32),
            pltpu.VMEM((T * B + 8,), jnp.int32),
            pltpu.VMEM((T * B + 8,), jnp.int32),
            pltpu.VMEM((16,), jnp.float32),
            pltpu.VMEM((16,), jnp.float32),
            pltpu.VMEM((16,), jnp.int32),
            pltpu.VMEM_SHARED((NS, 16), jnp.float32),
            pltpu.VMEM_SHARED((NS, 16), jnp.int32),
            pltpu.SemaphoreType.DMA,
        ],
    )
    return fn(tab, asr_w)


def kernel(asr_outputs, lm_emb, lm_proj):
    asr_w = _asr_prep(asr_outputs)
    tab = _lm_table(lm_emb, lm_proj)
    tab2 = tab.reshape(V * NS, VS)
    asrf = asr_w.reshape(T * V)
    fs, toks, pars = _beam_sc(tab2, asrf)
    return fs[:B], toks.reshape(T, B), pars.reshape(T, B)


# speedup vs baseline: 6.1555x; 6.1555x over previous
import jax
import jax.numpy as jnp
from jax import lax
from jax.experimental import pallas as pl
from jax.experimental.pallas import tpu as pltpu
from jax.experimental.pallas import tpu_sc as plsc

T = 512
V = 32768
B = 8
D = 64
W_ASR = 0.7
W_LM = 0.3
NEG = -1e30

NS = 16
VS = V // NS
NJ = VS // 16


def _asr_body(x_ref, o_ref):
    x = x_ref[...]
    m = jnp.max(x, axis=1, keepdims=True)
    sh = x - m
    lse = jnp.log(jnp.sum(jnp.exp(sh), axis=1, keepdims=True))
    o_ref[...] = W_ASR * (sh - lse)


def _asr_prep(asr):
    RB = 64
    return pl.pallas_call(
        _asr_body,
        grid=(T // RB,),
        in_specs=[pl.BlockSpec((RB, V), lambda i: (i, 0))],
        out_specs=pl.BlockSpec((RB, V), lambda i: (i, 0)),
        out_shape=jax.ShapeDtypeStruct((T, V), jnp.float32),
    )(asr)


def _tab_body(emb_ref, proj_ref, o_ref):
    logits = jnp.dot(emb_ref[...], proj_ref[...],
                     preferred_element_type=jnp.float32)
    m = jnp.max(logits, axis=1, keepdims=True)
    sh = logits - m
    lse = jnp.log(jnp.sum(jnp.exp(sh), axis=1, keepdims=True))
    o_ref[...] = W_LM * (sh - lse)


def _lm_table(emb, proj):
    RB = 64
    return pl.pallas_call(
        _tab_body,
        grid=(V // RB,),
        in_specs=[pl.BlockSpec((RB, D), lambda i: (i, 0)),
                  pl.BlockSpec((D, V), lambda i: (0, 0))],
        out_specs=pl.BlockSpec((RB, V), lambda i: (i, 0)),
        out_shape=jax.ShapeDtypeStruct((V, V), jnp.float32),
    )(emb, proj)


def _sc_body(tab_ref, asr_ref,
             fs_ref, tok_ref, par_ref, stage_ref,
             idx_v, rows_v, asr_v, wbuf, cval, cidx,
             allb, tokbuf, parbuf, fsbuf, pub,
             sem):
    wid = lax.axis_index("s")
    lane = lax.iota(jnp.int32, 16)
    lane_lt8 = lane < 8
    negv = jnp.full((16,), NEG, jnp.float32)
    zeroi = jnp.zeros((16,), jnp.int32)
    col0 = wid * VS

    score0 = jnp.where(lane == 0, jnp.float32(0.0), jnp.float32(NEG))
    tok0 = zeroi

    def step(t, carry):
        score, tok = carry

        idx_v[...] = tok * NS + wid
        lm_cp = pltpu.make_async_copy(tab_ref.at[idx_v], rows_v, sem)
        lm_cp.start()
        asr_cp = pltpu.make_async_copy(
            asr_ref.at[pl.ds((t * NS + wid) * VS, VS)], asr_v, sem)
        asr_cp.start()
        lm_cp.wait()
        asr_cp.wait()

        sbs = [score.at[jnp.full((16,), b, jnp.int32)]
               .get(mode="promise_in_bounds") for b in range(B)]

        def p1(j, m):
            a = asr_v[pl.ds(j * 16, 16)]
            for b in range(B):
                w = (a + rows_v[b, pl.ds(j * 16, 16)]) + sbs[b]
                wbuf[pl.ds(b * VS + j * 16, 16)] = w
                m = jnp.maximum(m, w)
            return m

        m = lax.fori_loop(0, NJ, p1, negv)

        msort, _ = plsc.sort_key_val(m, lane, descending=True)
        thr = (msort.at[jnp.full((16,), 7, jnp.int32)]
               .get(mode="promise_in_bounds"))

        off = jnp.int32(0)
        for b in range(B):
            def p2b(j, o, b=b):
                w = wbuf[pl.ds(b * VS + j * 16, 16)]
                sel = w >= thr
                iv = lane + (b * V + col0 + j * 16)
                seli = sel.astype(jnp.int32)
                csum = plsc.cumsum(seli)
                pos = o + (csum - seli)
                plsc.store_scatter(cval, [pos], w, mask=sel)
                plsc.store_scatter(cidx, [pos], iv, mask=sel)
                return o + jnp.max(csum)

            off = lax.fori_loop(0, NJ, p2b, off)
        plsc.store_scatter(cval, [off + lane], negv)

        def lmerge(cv, ci, nv_, ni_):
            rv = lax.rev(nv_, (0,))
            ri = lax.rev(ni_, (0,))
            keep = (cv > rv) | ((cv == rv) & (ci <= ri))
            mv = jnp.where(keep, cv, rv)
            mi = jnp.where(keep, ci, ri)
            i1, v1 = plsc.sort_key_val(mi, mv, descending=False)
            v2, i2 = plsc.sort_key_val(v1, i1, descending=True)
            return v2, i2

        def mrg(k, cur):
            cv, ci = cur
            vs, ivs = plsc.sort_key_val(cval[pl.ds(k * 16, 16)],
                                        cidx[pl.ds(k * 16, 16)],
                                        descending=True)
            return lmerge(cv, ci, vs, ivs)

        nv = (off + 15) >> 4
        cv, ci = lax.fori_loop(0, nv, mrg, (negv, zeroi))

        pub[pl.ds(0, 16)] = plsc.bitcast(cv, jnp.int32)
        pub[pl.ds(16, 16)] = ci
        pltpu.sync_copy(pub, stage_ref.at[pl.ds(wid * 32, 32)])
        plsc.subcore_barrier()
        rd = pltpu.make_async_copy(stage_ref, allb, sem)
        rd.start()
        rd.wait()
        plsc.subcore_barrier()

        def gmrg(k, cur):
            cv2, ci2 = cur
            vs = plsc.bitcast(allb[pl.ds(k * 32, 16)], jnp.float32)
            ivs = allb[pl.ds(k * 32 + 16, 16)]
            return lmerge(cv2, ci2, vs, ivs)

        gv, gi = lax.fori_loop(0, NS, gmrg, (negv, zeroi))

        par = lax.shift_right_arithmetic(gi, 15)
        tokn = jnp.bitwise_and(gi, V - 1)
        plsc.store_compressed(tokbuf.at[pl.ds(t * B, 16)], tokn, mask=lane_lt8)
        plsc.store_compressed(parbuf.at[pl.ds(t * B, 16)], par, mask=lane_lt8)
        return (gv, tokn)

    score, _ = lax.fori_loop(0, T, step, (score0, tok0))

    @pl.when(wid == 0)
    def _():
        fsbuf[...] = score
        pltpu.sync_copy(fsbuf, fs_ref)
        pltpu.sync_copy(tokbuf.at[pl.ds(0, T * B)], tok_ref)
        pltpu.sync_copy(parbuf.at[pl.ds(0, T * B)], par_ref)


def _beam_sc(tab, asr_w):
    mesh = plsc.VectorSubcoreMesh(core_axis_name="c", subcore_axis_name="s",
                                  num_cores=1)
    fn = pl.kernel(
        _sc_body,
        out_type=[
            jax.ShapeDtypeStruct((16,), jnp.float32),
            jax.ShapeDtypeStruct((T * B,), jnp.int32),
            jax.ShapeDtypeStruct((T * B,), jnp.int32),
            jax.ShapeDtypeStruct((NS * 32,), jnp.int32),
        ],
        mesh=mesh,
        compiler_params=pltpu.CompilerParams(needs_layout_passes=False),
        scratch_types=[
            pltpu.VMEM((16,), jnp.int32),
            pltpu.VMEM((16, VS), jnp.float32),
            pltpu.VMEM((VS,), jnp.float32),
            pltpu.VMEM((B * VS,), jnp.float32),
            pltpu.VMEM((B * VS + 16,), jnp.float32),
            pltpu.VMEM((B * VS + 16,), jnp.int32),
            pltpu.VMEM((NS * 32,), jnp.int32),
            pltpu.VMEM((T * B + 8,), jnp.int32),
            pltpu.VMEM((T * B + 8,), jnp.int32),
            pltpu.VMEM((16,), jnp.float32),
            pltpu.VMEM((32,), jnp.int32),
            pltpu.SemaphoreType.DMA,
        ],
    )
    return fn(tab, asr_w)[:3]


def kernel(asr_outputs, lm_emb, lm_proj):
    asr_w = _asr_prep(asr_outputs)
    tab = _lm_table(lm_emb, lm_proj)
    tab2 = tab.reshape(V * NS, VS)
    asrf = asr_w.reshape(T * V)
    fs, toks, pars = _beam_sc(tab2, asrf)
    return fs[:B], toks.reshape(T, B), pars.reshape(T, B)


# speedup vs baseline: 9.3780x; 1.5235x over previous
import jax
import jax.numpy as jnp
from jax import lax
from jax.experimental import pallas as pl
from jax.experimental.pallas import tpu as pltpu
from jax.experimental.pallas import tpu_sc as plsc

T = 512
V = 32768
B = 8
D = 64
W_ASR = 0.7
W_LM = 0.3
NEG = -1e30

NS = 16
VS = V // NS
NJ = VS // 16


def _asr_body(x_ref, o_ref):
    x = x_ref[...]
    m = jnp.max(x, axis=1, keepdims=True)
    sh = x - m
    lse = jnp.log(jnp.sum(jnp.exp(sh), axis=1, keepdims=True))
    o_ref[...] = W_ASR * (sh - lse)


def _asr_prep(asr):
    RB = 64
    return pl.pallas_call(
        _asr_body,
        grid=(T // RB,),
        in_specs=[pl.BlockSpec((RB, V), lambda i: (i, 0))],
        out_specs=pl.BlockSpec((RB, V), lambda i: (i, 0)),
        out_shape=jax.ShapeDtypeStruct((T, V), jnp.float32),
    )(asr)


def _tab_body(emb_ref, proj_ref, o_ref):
    logits = jnp.dot(emb_ref[...], proj_ref[...],
                     preferred_element_type=jnp.float32)
    m = jnp.max(logits, axis=1, keepdims=True)
    sh = logits - m
    lse = jnp.log(jnp.sum(jnp.exp(sh), axis=1, keepdims=True))
    o_ref[...] = W_LM * (sh - lse)


def _lm_table(emb, proj):
    RB = 128
    return pl.pallas_call(
        _tab_body,
        grid=(V // RB,),
        in_specs=[pl.BlockSpec((RB, D), lambda i: (i, 0)),
                  pl.BlockSpec((D, V), lambda i: (0, 0))],
        out_specs=pl.BlockSpec((RB, V), lambda i: (i, 0)),
        out_shape=jax.ShapeDtypeStruct((V, V), jnp.float32),
    )(emb, proj)


def _sc_body(tab_ref, asr_ref,
             fs_ref, tok_ref, par_ref, stage_ref,
             idx_v, rows_v, asr_v, wbuf, mbuf, cval, cidx,
             allb, tokbuf, parbuf, fsbuf, pub, jlist,
             sem):
    wid = lax.axis_index("s")
    lane = lax.iota(jnp.int32, 16)
    lane_lt8 = lane < 8
    negv = jnp.full((16,), NEG, jnp.float32)
    zeroi = jnp.zeros((16,), jnp.int32)
    col0 = wid * VS

    score0 = jnp.where(lane == 0, jnp.float32(0.0), jnp.float32(NEG))
    tok0 = zeroi

    def step(t, carry):
        score, tok = carry

        plsc.store_scatter(idx_v, [lane], tok * NS + wid, mask=lane_lt8)
        lm_cp = pltpu.make_async_copy(tab_ref.at[idx_v], rows_v, sem)
        lm_cp.start()
        asr_cp = pltpu.make_async_copy(
            asr_ref.at[pl.ds((t * NS + wid) * VS, VS)], asr_v, sem)
        asr_cp.start()
        lm_cp.wait()
        asr_cp.wait()

        sbs = [score.at[jnp.full((16,), b, jnp.int32)]
               .get(mode="promise_in_bounds") for b in range(B)]

        def p1(j, m):
            a = asr_v[pl.ds(j * 16, 16)]
            mj = None
            for b in range(B):
                w = (a + rows_v[b, pl.ds(j * 16, 16)]) + sbs[b]
                wbuf[pl.ds(b * VS + j * 16, 16)] = w
                mj = w if mj is None else jnp.maximum(mj, w)
            mbuf[pl.ds(j * 16, 16)] = mj
            return jnp.maximum(m, mj)

        m = lax.fori_loop(0, NJ, p1, negv)

        msort, _ = plsc.sort_key_val(m, lane, descending=True)
        thr = (msort.at[jnp.full((16,), 7, jnp.int32)]
               .get(mode="promise_in_bounds"))

        def jf(j, cnt):
            hit = jnp.any(mbuf[pl.ds(j * 16, 16)] >= thr)

            def yes(c):
                jlist[c] = j
                return c + 1

            return lax.cond(hit, yes, lambda c: c, cnt)

        njb = lax.fori_loop(0, NJ, jf, jnp.int32(0))

        off = jnp.int32(0)
        for b in range(B):
            def p2b(k, o, b=b):
                j = jlist[k]
                w = wbuf[pl.ds(b * VS + j * 16, 16)]
                sel = w >= thr
                iv = lane + (b * V + col0 + j * 16)
                seli = sel.astype(jnp.int32)
                csum = plsc.cumsum(seli)
                pos = o + (csum - seli)
                plsc.store_scatter(cval, [pos], w, mask=sel)
                plsc.store_scatter(cidx, [pos], iv, mask=sel)
                return o + jnp.max(csum)

            off = lax.fori_loop(0, njb, p2b, off)
        plsc.store_scatter(cval, [off + lane], negv)

        def lmerge(cv, ci, nv_, ni_):
            rv = lax.rev(nv_, (0,))
            ri = lax.rev(ni_, (0,))
            keep = (cv > rv) | ((cv == rv) & (ci <= ri))
            mv = jnp.where(keep, cv, rv)
            mi = jnp.where(keep, ci, ri)
            i1, v1 = plsc.sort_key_val(mi, mv, descending=False)
            v2, i2 = plsc.sort_key_val(v1, i1, descending=True)
            return v2, i2

        def mrg(k, cur):
            cv, ci = cur
            vs, ivs = plsc.sort_key_val(cval[pl.ds(k * 16, 16)],
                                        cidx[pl.ds(k * 16, 16)],
                                        descending=True)
            return lmerge(cv, ci, vs, ivs)

        nv = (off + 15) >> 4
        cv, ci = lax.fori_loop(0, nv, mrg, (negv, zeroi))

        pub[pl.ds(0, 16)] = plsc.bitcast(cv, jnp.int32)
        pub[pl.ds(16, 16)] = ci
        pltpu.sync_copy(pub, stage_ref.at[pl.ds(wid * 32, 32)])
        plsc.subcore_barrier()
        rd = pltpu.make_async_copy(stage_ref, allb, sem)
        rd.start()
        rd.wait()
        plsc.subcore_barrier()

        def gmrg(k, cur):
            cv2, ci2 = cur
            vs = plsc.bitcast(allb[pl.ds(k * 32, 16)], jnp.float32)
            ivs = allb[pl.ds(k * 32 + 16, 16)]
            return lmerge(cv2, ci2, vs, ivs)

        gv, gi = lax.fori_loop(0, NS, gmrg, (negv, zeroi))

        par = lax.shift_right_arithmetic(gi, 15)
        tokn = jnp.bitwise_and(gi, V - 1)
        plsc.store_compressed(tokbuf.at[pl.ds(t * B, 16)], tokn, mask=lane_lt8)
        plsc.store_compressed(parbuf.at[pl.ds(t * B, 16)], par, mask=lane_lt8)
        return (gv, tokn)

    score, _ = lax.fori_loop(0, T, step, (score0, tok0))

    @pl.when(wid == 0)
    def _():
        fsbuf[...] = score
        pltpu.sync_copy(fsbuf, fs_ref)
        pltpu.sync_copy(tokbuf.at[pl.ds(0, T * B)], tok_ref)
        pltpu.sync_copy(parbuf.at[pl.ds(0, T * B)], par_ref)


def _beam_sc(tab, asr_w):
    mesh = plsc.VectorSubcoreMesh(core_axis_name="c", subcore_axis_name="s",
                                  num_cores=1)
    fn = pl.kernel(
        _sc_body,
        out_type=[
            jax.ShapeDtypeStruct((16,), jnp.float32),
            jax.ShapeDtypeStruct((T * B,), jnp.int32),
            jax.ShapeDtypeStruct((T * B,), jnp.int32),
            jax.ShapeDtypeStruct((NS * 32,), jnp.int32),
        ],
        mesh=mesh,
        compiler_params=pltpu.CompilerParams(needs_layout_passes=False),
        scratch_types=[
            pltpu.VMEM((8,), jnp.int32),
            pltpu.VMEM((B, VS), jnp.float32),
            pltpu.VMEM((VS,), jnp.float32),
            pltpu.VMEM((B * VS,), jnp.float32),
            pltpu.VMEM((VS,), jnp.float32),
            pltpu.VMEM((B * VS + 16,), jnp.float32),
            pltpu.VMEM((B * VS + 16,), jnp.int32),
            pltpu.VMEM((NS * 32,), jnp.int32),
            pltpu.VMEM((T * B + 8,), jnp.int32),
            pltpu.VMEM((T * B + 8,), jnp.int32),
            pltpu.VMEM((16,), jnp.float32),
            pltpu.VMEM((32,), jnp.int32),
            pltpu.SMEM((NJ,), jnp.int32),
            pltpu.SemaphoreType.DMA,
        ],
    )
    return fn(tab, asr_w)[:3]


def kernel(asr_outputs, lm_emb, lm_proj):
    asr_w = _asr_prep(asr_outputs)
    tab = _lm_table(lm_emb, lm_proj)
    tab2 = tab.reshape(V * NS, VS)
    asrf = asr_w.reshape(T * V)
    fs, toks, pars = _beam_sc(tab2, asrf)
    return fs[:B], toks.reshape(T, B), pars.reshape(T, B)


# speedup vs baseline: 11.8893x; 1.2678x over previous
import jax
import jax.numpy as jnp
from jax import lax
from jax.experimental import pallas as pl
from jax.experimental.pallas import tpu as pltpu
from jax.experimental.pallas import tpu_sc as plsc

T = 512
V = 32768
B = 8
D = 64
W_ASR = 0.7
W_LM = 0.3
NEG = -1e30

NS = 16
VS = V // NS
NJ = VS // 16


def _asr_body(x_ref, o_ref):
    x = x_ref[...]
    m = jnp.max(x, axis=1, keepdims=True)
    sh = x - m
    lse = jnp.log(jnp.sum(jnp.exp(sh), axis=1, keepdims=True))
    o_ref[...] = W_ASR * (sh - lse)


def _asr_prep(asr):
    RB = 64
    return pl.pallas_call(
        _asr_body,
        grid=(T // RB,),
        in_specs=[pl.BlockSpec((RB, V), lambda i: (i, 0))],
        out_specs=pl.BlockSpec((RB, V), lambda i: (i, 0)),
        out_shape=jax.ShapeDtypeStruct((T, V), jnp.float32),
    )(asr)


def _tab_body(emb_ref, proj_ref, o_ref):
    logits = jnp.dot(emb_ref[...], proj_ref[...],
                     preferred_element_type=jnp.float32)
    m = jnp.max(logits, axis=1, keepdims=True)
    sh = logits - m
    lse = jnp.log(jnp.sum(jnp.exp(sh), axis=1, keepdims=True))
    o_ref[...] = W_LM * (sh - lse)


def _lm_table(emb, proj):
    RB = 128
    return pl.pallas_call(
        _tab_body,
        grid=(V // RB,),
        in_specs=[pl.BlockSpec((RB, D), lambda i: (i, 0)),
                  pl.BlockSpec((D, V), lambda i: (0, 0))],
        out_specs=pl.BlockSpec((RB, V), lambda i: (i, 0)),
        out_shape=jax.ShapeDtypeStruct((V, V), jnp.float32),
    )(emb, proj)


def _sc_body(tab_ref, asr_ref,
             fs_ref, tok_ref, par_ref, stage_ref,
             idx_v, rows_v, asr_v, wbuf, mbuf, cval, cidx,
             allb, tokbuf, parbuf, fsbuf, pub, jlist,
             sem):
    wid = lax.axis_index("s")
    lane = lax.iota(jnp.int32, 16)
    lane_lt8 = lane < 8
    negv = jnp.full((16,), NEG, jnp.float32)
    zeroi = jnp.zeros((16,), jnp.int32)
    col0 = wid * VS

    score0 = jnp.where(lane == 0, jnp.float32(0.0), jnp.float32(NEG))
    tok0 = zeroi

    def step(t, carry):
        score, tok = carry

        plsc.store_scatter(idx_v, [lane], tok * NS + wid, mask=lane_lt8)
        lm_cp = pltpu.make_async_copy(tab_ref.at[idx_v], rows_v, sem)
        lm_cp.start()
        asr_cp = pltpu.make_async_copy(
            asr_ref.at[pl.ds((t * NS + wid) * VS, VS)], asr_v, sem)
        asr_cp.start()
        lm_cp.wait()
        asr_cp.wait()

        sbs = [score.at[jnp.full((16,), b, jnp.int32)]
               .get(mode="promise_in_bounds") for b in range(B)]

        @plsc.parallel_loop(0, NJ, step=1, unroll=4, carry=negv)
        def p1m(j, m):
            a = asr_v[pl.ds(j * 16, 16)]
            mj = None
            for b in range(B):
                w = (a + rows_v[b, pl.ds(j * 16, 16)]) + sbs[b]
                wbuf[pl.ds(b * VS + j * 16, 16)] = w
                mj = w if mj is None else jnp.maximum(mj, w)
            mbuf[pl.ds(j * 16, 16)] = mj
            return jnp.maximum(m, mj)

        m = p1m

        msort, _ = plsc.sort_key_val(m, lane, descending=True)
        thr = (msort.at[jnp.full((16,), 7, jnp.int32)]
               .get(mode="promise_in_bounds"))

        def jf(j, cnt):
            hit = jnp.any(mbuf[pl.ds(j * 16, 16)] >= thr)

            def yes(c):
                jlist[c] = j
                return c + 1

            return lax.cond(hit, yes, lambda c: c, cnt)

        njb = lax.fori_loop(0, NJ, jf, jnp.int32(0))

        off = jnp.int32(0)
        for b in range(B):
            def p2b(k, o, b=b):
                j = jlist[k]
                w = wbuf[pl.ds(b * VS + j * 16, 16)]
                sel = w >= thr
                iv = lane + (b * V + col0 + j * 16)
                seli = sel.astype(jnp.int32)
                csum = plsc.cumsum(seli)
                pos = o + (csum - seli)
                plsc.store_scatter(cval, [pos], w, mask=sel)
                plsc.store_scatter(cidx, [pos], iv, mask=sel)
                return o + jnp.max(csum)

            off = lax.fori_loop(0, njb, p2b, off)
        plsc.store_scatter(cval, [off + lane], negv)

        def lmerge(cv, ci, nv_, ni_):
            rv = lax.rev(nv_, (0,))
            ri = lax.rev(ni_, (0,))
            keep = (cv > rv) | ((cv == rv) & (ci <= ri))
            mv = jnp.where(keep, cv, rv)
            mi = jnp.where(keep, ci, ri)
            i1, v1 = plsc.sort_key_val(mi, mv, descending=False)
            v2, i2 = plsc.sort_key_val(v1, i1, descending=True)
            return v2, i2

        def mrg(k, cur):
            cv, ci = cur
            vs, ivs = plsc.sort_key_val(cval[pl.ds(k * 16, 16)],
                                        cidx[pl.ds(k * 16, 16)],
                                        descending=True)
            return lmerge(cv, ci, vs, ivs)

        nv = (off + 15) >> 4
        cv, ci = lax.fori_loop(0, nv, mrg, (negv, zeroi))

        pub[pl.ds(0, 16)] = plsc.bitcast(cv, jnp.int32)
        pub[pl.ds(16, 16)] = ci
        pltpu.sync_copy(pub, stage_ref.at[pl.ds(wid * 32, 32)])
        plsc.subcore_barrier()
        rd = pltpu.make_async_copy(stage_ref, allb, sem)
        rd.start()
        rd.wait()
        plsc.subcore_barrier()

        def gmrg(k, cur):
            cv2, ci2 = cur
            vs = plsc.bitcast(allb[pl.ds(k * 32, 16)], jnp.float32)
            ivs = allb[pl.ds(k * 32 + 16, 16)]
            return lmerge(cv2, ci2, vs, ivs)

        gv, gi = lax.fori_loop(0, NS, gmrg, (negv, zeroi))

        par = lax.shift_right_arithmetic(gi, 15)
        tokn = jnp.bitwise_and(gi, V - 1)
        plsc.store_compressed(tokbuf.at[pl.ds(t * B, 16)], tokn, mask=lane_lt8)
        plsc.store_compressed(parbuf.at[pl.ds(t * B, 16)], par, mask=lane_lt8)
        return (gv, tokn)

    score, _ = lax.fori_loop(0, T, step, (score0, tok0))

    @pl.when(wid == 0)
    def _():
        fsbuf[...] = score
        pltpu.sync_copy(fsbuf, fs_ref)
        pltpu.sync_copy(tokbuf.at[pl.ds(0, T * B)], tok_ref)
        pltpu.sync_copy(parbuf.at[pl.ds(0, T * B)], par_ref)


def _beam_sc(tab, asr_w):
    mesh = plsc.VectorSubcoreMesh(core_axis_name="c", subcore_axis_name="s",
                                  num_cores=1)
    fn = pl.kernel(
        _sc_body,
        out_type=[
            jax.ShapeDtypeStruct((16,), jnp.float32),
            jax.ShapeDtypeStruct((T * B,), jnp.int32),
            jax.ShapeDtypeStruct((T * B,), jnp.int32),
            jax.ShapeDtypeStruct((NS * 32,), jnp.int32),
        ],
        mesh=mesh,
        compiler_params=pltpu.CompilerParams(needs_layout_passes=False),
        scratch_types=[
            pltpu.VMEM((8,), jnp.int32),
            pltpu.VMEM((B, VS), jnp.float32),
            pltpu.VMEM((VS,), jnp.float32),
            pltpu.VMEM((B * VS,), jnp.float32),
            pltpu.VMEM((VS,), jnp.float32),
            pltpu.VMEM((B * VS + 16,), jnp.float32),
            pltpu.VMEM((B * VS + 16,), jnp.int32),
            pltpu.VMEM((NS * 32,), jnp.int32),
            pltpu.VMEM((T * B + 8,), jnp.int32),
            pltpu.VMEM((T * B + 8,), jnp.int32),
            pltpu.VMEM((16,), jnp.float32),
            pltpu.VMEM((32,), jnp.int32),
            pltpu.SMEM((NJ,), jnp.int32),
            pltpu.SemaphoreType.DMA,
        ],
    )
    return fn(tab, asr_w)[:3]


def kernel(asr_outputs, lm_emb, lm_proj):
    asr_w = _asr_prep(asr_outputs)
    tab = _lm_table(lm_emb, lm_proj)
    tab2 = tab.reshape(V * NS, VS)
    asrf = asr_w.reshape(T * V)
    fs, toks, pars = _beam_sc(tab2, asrf)
    return fs[:B], toks.reshape(T, B), pars.reshape(T, B)


# speedup vs baseline: 12.0107x; 1.0102x over previous
import jax
import jax.numpy as jnp
from jax import lax
from jax.experimental import pallas as pl
from jax.experimental.pallas import tpu as pltpu
from jax.experimental.pallas import tpu_sc as plsc

T = 512
V = 32768
B = 8
D = 64
W_ASR = 0.7
W_LM = 0.3
NEG = -1e30

NS = 16
VS = V // NS
NJ = VS // 16


def _asr_body(x_ref, o_ref):
    x = x_ref[...]
    m = jnp.max(x, axis=1, keepdims=True)
    sh = x - m
    lse = jnp.log(jnp.sum(jnp.exp(sh), axis=1, keepdims=True))
    o_ref[...] = W_ASR * (sh - lse)


def _asr_prep(asr):
    RB = 64
    return pl.pallas_call(
        _asr_body,
        grid=(T // RB,),
        in_specs=[pl.BlockSpec((RB, V), lambda i: (i, 0))],
        out_specs=pl.BlockSpec((RB, V), lambda i: (i, 0)),
        out_shape=jax.ShapeDtypeStruct((T, V), jnp.float32),
    )(asr)


def _tab_body(emb_ref, proj_ref, o_ref):
    logits = jnp.dot(emb_ref[...], proj_ref[...],
                     preferred_element_type=jnp.float32)
    m = jnp.max(logits, axis=1, keepdims=True)
    sh = logits - m
    lse = jnp.log(jnp.sum(jnp.exp(sh), axis=1, keepdims=True))
    o_ref[...] = W_LM * (sh - lse)


def _lm_table(emb, proj):
    RB = 128
    return pl.pallas_call(
        _tab_body,
        grid=(V // RB,),
        in_specs=[pl.BlockSpec((RB, D), lambda i: (i, 0)),
                  pl.BlockSpec((D, V), lambda i: (0, 0))],
        out_specs=pl.BlockSpec((RB, V), lambda i: (i, 0)),
        out_shape=jax.ShapeDtypeStruct((V, V), jnp.float32),
    )(emb, proj)


def _sc_body(tab_ref, asr_ref,
             fs_ref, tok_ref, par_ref, stage_ref,
             idx_v, rows_v, rows_w, asr_v, wbuf, mbuf, cval, cidx,
             allb, tokbuf, parbuf, fsbuf, pub, jlist,
             sem, sem2):
    wid = lax.axis_index("s")
    lane = lax.iota(jnp.int32, 16)
    lane_lt8 = lane < 8
    negv = jnp.full((16,), NEG, jnp.float32)
    zeroi = jnp.zeros((16,), jnp.int32)
    col0 = wid * VS

    score0 = jnp.where(lane == 0, jnp.float32(0.0), jnp.float32(NEG))
    tok0 = zeroi

    def step(t, carry):
        score, tok = carry

        pos8 = jnp.where(lane < 4, lane, lane + 4)
        plsc.store_scatter(idx_v, [pos8], tok * NS + wid, mask=lane_lt8)
        lm_a = pltpu.make_async_copy(tab_ref.at[idx_v.at[pl.ds(0, 4)]],
                                     rows_v, sem)
        lm_a.start()
        lm_b = pltpu.make_async_copy(tab_ref.at[idx_v.at[pl.ds(8, 4)]],
                                     rows_w, sem2)
        lm_b.start()
        asr_cp = pltpu.make_async_copy(
            asr_ref.at[pl.ds((t * NS + wid) * VS, VS)], asr_v, sem)
        asr_cp.start()
        lm_a.wait()
        asr_cp.wait()

        sbs = [score.at[jnp.full((16,), b, jnp.int32)]
               .get(mode="promise_in_bounds") for b in range(B)]

        @plsc.parallel_loop(0, NJ, step=1, unroll=4, carry=negv)
        def p1a(j, m):
            a = asr_v[pl.ds(j * 16, 16)]
            mj = None
            for b in range(4):
                w = (a + rows_v[b, pl.ds(j * 16, 16)]) + sbs[b]
                wbuf[pl.ds(b * VS + j * 16, 16)] = w
                mj = w if mj is None else jnp.maximum(mj, w)
            mbuf[pl.ds(j * 16, 16)] = mj
            return jnp.maximum(m, mj)

        lm_b.wait()

        @plsc.parallel_loop(0, NJ, step=1, unroll=4, carry=p1a)
        def p1b(j, m):
            a = asr_v[pl.ds(j * 16, 16)]
            mj = None
            for b in range(4, B):
                w = (a + rows_w[b - 4, pl.ds(j * 16, 16)]) + sbs[b]
                wbuf[pl.ds(b * VS + j * 16, 16)] = w
                mj = w if mj is None else jnp.maximum(mj, w)
            mj = jnp.maximum(mj, mbuf[pl.ds(j * 16, 16)])
            mbuf[pl.ds(j * 16, 16)] = mj
            return jnp.maximum(m, mj)

        m = p1b

        msort, _ = plsc.sort_key_val(m, lane, descending=True)
        thr = (msort.at[jnp.full((16,), 7, jnp.int32)]
               .get(mode="promise_in_bounds"))

        def jf(j, cnt):
            hit = jnp.any(mbuf[pl.ds(j * 16, 16)] >= thr)

            def yes(c):
                jlist[c] = j
                return c + 1

            return lax.cond(hit, yes, lambda c: c, cnt)

        njb = lax.fori_loop(0, NJ, jf, jnp.int32(0))

        off = jnp.int32(0)
        for b in range(B):
            def p2b(k, o, b=b):
                j = jlist[k]
                w = wbuf[pl.ds(b * VS + j * 16, 16)]
                sel = w >= thr
                iv = lane + (b * V + col0 + j * 16)
                seli = sel.astype(jnp.int32)
                csum = plsc.cumsum(seli)
                pos = o + (csum - seli)
                plsc.store_scatter(cval, [pos], w, mask=sel)
                plsc.store_scatter(cidx, [pos], iv, mask=sel)
                return o + jnp.max(csum)

            off = lax.fori_loop(0, njb, p2b, off)
        plsc.store_scatter(cval, [off + lane], negv)

        def lmerge(cv, ci, nv_, ni_):
            rv = lax.rev(nv_, (0,))
            ri = lax.rev(ni_, (0,))
            keep = (cv > rv) | ((cv == rv) & (ci <= ri))
            mv = jnp.where(keep, cv, rv)
            mi = jnp.where(keep, ci, ri)
            i1, v1 = plsc.sort_key_val(mi, mv, descending=False)
            v2, i2 = plsc.sort_key_val(v1, i1, descending=True)
            return v2, i2

        def mrg(k, cur):
            cv, ci = cur
            vs, ivs = plsc.sort_key_val(cval[pl.ds(k * 16, 16)],
                                        cidx[pl.ds(k * 16, 16)],
                                        descending=True)
            return lmerge(cv, ci, vs, ivs)

        nv = (off + 15) >> 4
        cv, ci = lax.fori_loop(0, nv, mrg, (negv, zeroi))

        pub[pl.ds(0, 16)] = plsc.bitcast(cv, jnp.int32)
        pub[pl.ds(16, 16)] = ci
        pltpu.sync_copy(pub, stage_ref.at[pl.ds(wid * 32, 32)])
        plsc.subcore_barrier()
        rd = pltpu.make_async_copy(stage_ref, allb, sem)
        rd.start()
        rd.wait()
        plsc.subcore_barrier()

        def gmrg(k, cur):
            cv2, ci2 = cur
            vs = plsc.bitcast(allb[pl.ds(k * 32, 16)], jnp.float32)
            ivs = allb[pl.ds(k * 32 + 16, 16)]
            return lmerge(cv2, ci2, vs, ivs)

        gv, gi = lax.fori_loop(0, NS, gmrg, (negv, zeroi))

        par = lax.shift_right_arithmetic(gi, 15)
        tokn = jnp.bitwise_and(gi, V - 1)
        plsc.store_compressed(tokbuf.at[pl.ds(t * B, 16)], tokn, mask=lane_lt8)
        plsc.store_compressed(parbuf.at[pl.ds(t * B, 16)], par, mask=lane_lt8)
        return (gv, tokn)

    score, _ = lax.fori_loop(0, T, step, (score0, tok0))

    @pl.when(wid == 0)
    def _():
        fsbuf[...] = score
        pltpu.sync_copy(fsbuf, fs_ref)
        pltpu.sync_copy(tokbuf.at[pl.ds(0, T * B)], tok_ref)
        pltpu.sync_copy(parbuf.at[pl.ds(0, T * B)], par_ref)


def _beam_sc(tab, asr_w):
    mesh = plsc.VectorSubcoreMesh(core_axis_name="c", subcore_axis_name="s",
                                  num_cores=1)
    fn = pl.kernel(
        _sc_body,
        out_type=[
            jax.ShapeDtypeStruct((16,), jnp.float32),
            jax.ShapeDtypeStruct((T * B,), jnp.int32),
            jax.ShapeDtypeStruct((T * B,), jnp.int32),
            jax.ShapeDtypeStruct((NS * 32,), jnp.int32),
        ],
        mesh=mesh,
        compiler_params=pltpu.CompilerParams(needs_layout_passes=False),
        scratch_types=[
            pltpu.VMEM((16,), jnp.int32),
            pltpu.VMEM((4, VS), jnp.float32),
            pltpu.VMEM((4, VS), jnp.float32),
            pltpu.VMEM((VS,), jnp.float32),
            pltpu.VMEM((B * VS,), jnp.float32),
            pltpu.VMEM((VS,), jnp.float32),
            pltpu.VMEM((B * VS + 16,), jnp.float32),
            pltpu.VMEM((B * VS + 16,), jnp.int32),
            pltpu.VMEM((NS * 32,), jnp.int32),
            pltpu.VMEM((T * B + 8,), jnp.int32),
            pltpu.VMEM((T * B + 8,), jnp.int32),
            pltpu.VMEM((16,), jnp.float32),
            pltpu.VMEM((32,), jnp.int32),
            pltpu.SMEM((NJ,), jnp.int32),
            pltpu.SemaphoreType.DMA,
            pltpu.SemaphoreType.DMA,
        ],
    )
    return fn(tab, asr_w)[:3]


def kernel(asr_outputs, lm_emb, lm_proj):
    asr_w = _asr_prep(asr_outputs)
    tab = _lm_table(lm_emb, lm_proj)
    tab2 = tab.reshape(V * NS, VS)
    asrf = asr_w.reshape(T * V)
    fs, toks, pars = _beam_sc(tab2, asrf)
    return fs[:B], toks.reshape(T, B), pars.reshape(T, B)
